# baseline (speedup 1.0000x reference)
import jax
import jax.numpy as jnp
from jax import lax
from jax.experimental import pallas as pl
from jax.experimental.pallas import tpu as pltpu
from jax.experimental.pallas import tpu_sc as plsc

NC = 2
NS = 16
NW = NC * NS
L = 16

D = 1024
K = 8
NBUF = 4


def _sc_body(x_hbm, idx_hbm, tab_hbm, out_hbm, idx_v, *bufs):
    rows = bufs[0:NBUF]
    xb = bufs[NBUF:2 * NBUF]
    sg = bufs[2 * NBUF:3 * NBUF]
    sx = bufs[3 * NBUF:4 * NBUF]
    so = bufs[4 * NBUF:5 * NBUF]

    b_per_w = idx_v.shape[0]
    n_chunks = b_per_w // K
    wid = lax.axis_index("s") * NC + lax.axis_index("c")
    base = wid * b_per_w

    pltpu.sync_copy(idx_hbm.at[pl.ds(base, b_per_w)], idx_v)

    def issue_loads(c, p):
        dx = pltpu.async_copy(x_hbm.at[pl.ds(base + c * K, K)], xb[p], sx[p])
        dg = pltpu.async_copy(tab_hbm.at[idx_v.at[pl.ds(c * K, K)]],
                              rows[p], sg[p])
        return dx, dg

    def add_chunk(p):
        @plsc.parallel_loop(0, K)
        def _r(r):
            for c in range(D // L):
                xb[p][r, pl.ds(c * L, L)] = (
                    xb[p][r, pl.ds(c * L, L)] + rows[p][r, pl.ds(c * L, L)]
                )

    def drain_store(p):
        pltpu.make_async_copy(x_hbm.at[pl.ds(base, K)], xb[p], so[p]).wait()

    @pl.loop(0, n_chunks, step=NBUF)
    def _grp(g):
        @pl.when(g > 0)
        def _():
            for p in range(NBUF):
                drain_store(p)

        ds = [issue_loads(g + p, p) for p in range(NBUF)]
        for p in range(NBUF):
            dx, dg = ds[p]
            dx.wait()
            dg.wait()
            add_chunk(p)
            pltpu.async_copy(xb[p], out_hbm.at[pl.ds(base + (g + p) * K, K)],
                             so[p])

    for p in range(NBUF):
        drain_store(p)


def _sc_call(x2, idx, tab):
    B = x2.shape[0]
    b_per_w = B // NW
    mesh = plsc.VectorSubcoreMesh(core_axis_name="c", subcore_axis_name="s")
    k = pl.kernel(
        _sc_body,
        out_type=jax.ShapeDtypeStruct((B, D), jnp.float32),
        mesh=mesh,
        scratch_types=(
            [pltpu.VMEM((b_per_w,), jnp.int32)]
            + [pltpu.VMEM((K, D), jnp.float32) for _ in range(2 * NBUF)]
            + [pltpu.SemaphoreType.DMA for _ in range(3 * NBUF)]
        ),
    )
    return k(x2, idx, tab)


@jax.jit
def kernel(x, n, pos_table):
    b, s, d = x.shape
    x2 = x.reshape(b * s, d)
    idx = n.reshape(b * s).astype(jnp.int32)
    out = _sc_call(x2, idx, pos_table)
    return out.reshape(b, s, d)
